# Initial kernel scaffold; baseline (speedup 1.0000x reference)
import jax
import jax.numpy as jnp
from jax.experimental import pallas as pl


def kernel(x, edge_index, first_nodes_idx, ln1_g, ln1_b, Wg, bg, ln2_g, ln2_b, W1, b1, W2, b2):
    raise NotImplementedError("write your pallas kernel here")


# speedup vs baseline: 1.8423x; 1.8423x over previous
import functools

import jax
import jax.numpy as jnp
from jax import lax
from jax.experimental import pallas as pl
from jax.experimental.pallas import tpu as pltpu
from jax.experimental.pallas import tpu_sc as plsc

NODES = 10000
EDGES = 160000
FEAT = 256
HALF = 128
BATCH = 8
LAYERS = 2

NC = 2
NS = 16
EPT = EDGES // NS
K = 80
NCHUNK = EPT // K
NSPLIT = 2
SROWS = NODES // NSPLIT
DUMP = 64
ACCR = SROWS + DUMP
PSTRIDE = 312
PLEFT = SROWS - NS * PSTRIDE
PLEFT0 = NS * PSTRIDE
NPAD = 10240
RED = NPAD // NS


def _zero_1d(ref, n):
    def body(i, _):
        ref[pl.ds(i * 16, 16)] = jnp.zeros((16,), ref.dtype)
        return None

    lax.fori_loop(0, n // 16, body, None)


def _zero_2d(ref, nrows, ncols):
    cpl = ncols // 16

    def body(i, _):
        ref[i // cpl, pl.ds((i % cpl) * 16, 16)] = jnp.zeros((16,), ref.dtype)
        return None

    lax.fori_loop(0, nrows * cpl, body, None)


@functools.lru_cache(maxsize=None)
def _sc_kernels():
    mesh = plsc.VectorSubcoreMesh(
        core_axis_name="c", subcore_axis_name="s",
        num_cores=NC, num_subcores=NS)

    @functools.partial(
        pl.kernel,
        out_type=jax.ShapeDtypeStruct((2 * NPAD,), jnp.float32),
        mesh=mesh,
        scratch_types=[
            pltpu.VMEM((K,), jnp.int32),
            pltpu.VMEM((NPAD,), jnp.float32),
            pltpu.VMEM((RED,), jnp.float32),
            pltpu.VMEM((RED,), jnp.float32),
            pltpu.VMEM_SHARED((NS, 1, NPAD), jnp.float32),
        ],
        compiler_params=pltpu.CompilerParams(needs_layout_passes=False),
    )
    def deg_kernel(edge_ref, out_ref, idx_v, hist_v, tmp_v, red_v, spm):
        c = lax.axis_index("c")
        s = lax.axis_index("s")

        _zero_1d(hist_v, NPAD)
        ones16 = jnp.ones((16,), jnp.float32)

        def chunk(k, _):
            base = c * EDGES + s * EPT + k * K
            pltpu.sync_copy(edge_ref.at[pl.ds(base, K)], idx_v)

            def upd(j, _):
                idx16 = idx_v[pl.ds(j * 16, 16)]
                plsc.addupdate_scatter(hist_v, [idx16], ones16)
                return None

            lax.fori_loop(0, K // 16, upd, None)
            return None

        lax.fori_loop(0, NCHUNK, chunk, None)
        pltpu.sync_copy(hist_v, spm.at[s, 0])
        plsc.subcore_barrier()

        _zero_1d(red_v, RED)
        for t in range(NS):
            pltpu.sync_copy(spm.at[t, 0, pl.ds(s * RED, RED)], tmp_v)

            def acc(j, _):
                sl = pl.ds(j * 16, 16)
                red_v[sl] = red_v[sl] + tmp_v[sl]
                return None

            lax.fori_loop(0, RED // 16, acc, None)
        pltpu.sync_copy(red_v, out_ref.at[pl.ds(c * NPAD + s * RED, RED)])

    @functools.partial(
        pl.kernel,
        out_type=jax.ShapeDtypeStruct((2 * NODES, HALF), jnp.float32),
        mesh=mesh,
        scratch_types=[
            pltpu.VMEM((K,), jnp.int32),
            pltpu.VMEM((K,), jnp.int32),
            pltpu.VMEM((K, HALF), jnp.float32),
            pltpu.VMEM((PSTRIDE, HALF), jnp.float32),
            pltpu.VMEM_SHARED((ACCR, HALF), jnp.float32),
            pltpu.SemaphoreType.DMA,
        ],
    )
    def scatter_kernel(m_ref, edge_ref, out_ref, src_v, dst_v, rows_v,
                       stage_v, acc, sem):
        c = lax.axis_index("c")
        s = lax.axis_index("s")

        def addoff(j, _):
            sl = pl.ds(j * 16, 16)
            src_v[sl] = src_v[sl] + c * NODES
            return None

        for p in range(NSPLIT):
            _zero_2d(stage_v, PSTRIDE, HALF)
            pltpu.sync_copy(stage_v, acc.at[pl.ds(s * PSTRIDE, PSTRIDE)])

            @pl.when(s == 0)
            def _():
                pltpu.sync_copy(stage_v.at[pl.ds(0, PLEFT)],
                                acc.at[pl.ds(PLEFT0, PLEFT)])

            plsc.subcore_barrier()

            def fixdst(j, _):
                sl = pl.ds(j * 16, 16)
                d = dst_v[sl]
                t = d - p * SROWS
                ok = (t >= 0) & (t < SROWS)
                dst_v[sl] = jnp.where(ok, t, SROWS + (d & (DUMP - 1)))
                return None

            def chunk(k, _):
                base = s * EPT + k * K
                pltpu.sync_copy(edge_ref.at[pl.ds(base, K)], src_v)
                pltpu.sync_copy(edge_ref.at[pl.ds(EDGES + base, K)], dst_v)
                lax.fori_loop(0, K // 16, addoff, None)
                lax.fori_loop(0, K // 16, fixdst, None)
                pltpu.async_copy(m_ref.at[src_v], rows_v, sem).wait()
                pltpu.sync_copy(rows_v, acc.at[dst_v], add=True)
                return None

            lax.fori_loop(0, NCHUNK, chunk, None)
            plsc.subcore_barrier()
            obase = c * NODES + p * SROWS
            pltpu.sync_copy(acc.at[pl.ds(s * PSTRIDE, PSTRIDE)], stage_v)
            pltpu.sync_copy(stage_v,
                            out_ref.at[pl.ds(obase + s * PSTRIDE, PSTRIDE)])

            @pl.when(s == 0)
            def _():
                pltpu.sync_copy(acc.at[pl.ds(PLEFT0, PLEFT)],
                                stage_v.at[pl.ds(0, PLEFT)])
                pltpu.sync_copy(stage_v.at[pl.ds(0, PLEFT)],
                                out_ref.at[pl.ds(obase + PLEFT0, PLEFT)])

    return deg_kernel, scatter_kernel


R = 1000
GRID = NODES // R


def _ln(xb, g, b):
    mu = jnp.mean(xb, axis=-1, keepdims=True)
    xc = xb - mu
    var = jnp.mean(xc * xc, axis=-1, keepdims=True)
    return xc * lax.rsqrt(var + 1e-5) * g + b


def _inv_sqrt_deg(deg_blk):
    return lax.rsqrt(jnp.maximum(deg_blk, 1.0))


_VEC_SPEC = pl.BlockSpec((1, FEAT), lambda i: (0, 0))
_DEG_SPEC = pl.BlockSpec((R, 1), lambda i: (i, 0))
_MH_SPEC = pl.BlockSpec((2, R, HALF), lambda i: (0, i, 0))
_MH_SHAPE = jax.ShapeDtypeStruct((2, NODES, HALF), jnp.float32)


def _store_halves(m, m_ref):
    m_ref[0] = m[:, :HALF]
    m_ref[1] = m[:, HALF:]


def _tc_pre_body(x_ref, od_ref, g1_ref, b1_ref, m_ref):
    hn = _ln(x_ref[...], g1_ref[...], b1_ref[...])
    _store_halves(hn * _inv_sqrt_deg(od_ref[...]), m_ref)


_tc_pre = pl.pallas_call(
    _tc_pre_body,
    grid=(GRID,),
    in_specs=[
        pl.BlockSpec((R, FEAT), lambda i: (i, 0)),
        _DEG_SPEC,
        _VEC_SPEC,
        _VEC_SPEC,
    ],
    out_specs=_MH_SPEC,
    out_shape=_MH_SHAPE,
)


def _tc_mid_body(agg_ref, ind_ref, od_ref, wg_ref, bg_ref, g2_ref, b2_ref,
                 w1_ref, b1f_ref, w2_ref, b2f_ref, g1n_ref, b1n_ref,
                 m_ref, h_ref):
    a = jnp.concatenate([agg_ref[0], agg_ref[1]], axis=-1)
    a = a * _inv_sqrt_deg(ind_ref[...])
    h = jnp.dot(a, wg_ref[...], preferred_element_type=jnp.float32)
    h = h + bg_ref[...]
    hn2 = _ln(h, g2_ref[...], b2_ref[...])
    t = jnp.dot(hn2, w1_ref[...], preferred_element_type=jnp.float32)
    t = jnp.maximum(t + b1f_ref[...], 0.0)
    ff = jnp.dot(t, w2_ref[...], preferred_element_type=jnp.float32)
    h = ff + b2f_ref[...] + h
    h_ref[...] = h
    hn = _ln(h, g1n_ref[...], b1n_ref[...])
    _store_halves(hn * _inv_sqrt_deg(od_ref[...]), m_ref)


_tc_mid = pl.pallas_call(
    _tc_mid_body,
    grid=(GRID,),
    in_specs=[
        _MH_SPEC,
        _DEG_SPEC,
        _DEG_SPEC,
        pl.BlockSpec((FEAT, FEAT), lambda i: (0, 0)),
        _VEC_SPEC,
        _VEC_SPEC,
        _VEC_SPEC,
        pl.BlockSpec((FEAT, 2 * FEAT), lambda i: (0, 0)),
        pl.BlockSpec((1, 2 * FEAT), lambda i: (0, 0)),
        pl.BlockSpec((2 * FEAT, FEAT), lambda i: (0, 0)),
        _VEC_SPEC,
        _VEC_SPEC,
        _VEC_SPEC,
    ],
    out_specs=[_MH_SPEC, pl.BlockSpec((R, FEAT), lambda i: (i, 0))],
    out_shape=[_MH_SHAPE, jax.ShapeDtypeStruct((NODES, FEAT), jnp.float32)],
)


def _tc_read_body(h_ref, fni_ref, out_ref):
    i = pl.program_id(0)
    rows = lax.broadcasted_iota(jnp.int32, (BATCH, R), 1) + i * R
    onehot = (rows == fni_ref[...]).astype(jnp.float32)
    contrib = jnp.dot(onehot, h_ref[...], preferred_element_type=jnp.float32)

    @pl.when(i == 0)
    def _():
        out_ref[...] = contrib

    @pl.when(i > 0)
    def _():
        out_ref[...] += contrib


_tc_read = pl.pallas_call(
    _tc_read_body,
    grid=(GRID,),
    in_specs=[
        pl.BlockSpec((R, FEAT), lambda i: (i, 0)),
        pl.BlockSpec((BATCH, 1), lambda i: (0, 0)),
    ],
    out_specs=pl.BlockSpec((BATCH, FEAT), lambda i: (0, 0)),
    out_shape=jax.ShapeDtypeStruct((BATCH, FEAT), jnp.float32),
)


def kernel(x, edge_index, first_nodes_idx, ln1_g, ln1_b, Wg, bg, ln2_g,
           ln2_b, W1, b1, W2, b2):
    deg_kernel, scatter_kernel = _sc_kernels()
    edge_flat = edge_index.reshape(2 * EDGES)
    deg = deg_kernel(edge_flat)
    od = deg[:NODES].reshape(NODES, 1)
    ind = deg[NPAD:NPAD + NODES].reshape(NODES, 1)

    m0 = _tc_pre(x, od, ln1_g[0:1], ln1_b[0:1])

    ln1n_g = jnp.roll(ln1_g, -1, axis=0)
    ln1n_b = jnp.roll(ln1_b, -1, axis=0)

    limit = LAYERS + (edge_flat[0] >> 31)

    def cond(st):
        return st[0] < limit

    def body(st):
        i, m, _ = st
        agg = scatter_kernel(m.reshape(2 * NODES, HALF), edge_flat)
        m_next, h = _tc_mid(
            agg.reshape(2, NODES, HALF), ind, od, Wg[i],
            bg[i].reshape(1, FEAT), ln2_g[i].reshape(1, FEAT),
            ln2_b[i].reshape(1, FEAT), W1[i], b1[i].reshape(1, 2 * FEAT),
            W2[i], b2[i].reshape(1, FEAT), ln1n_g[i].reshape(1, FEAT),
            ln1n_b[i].reshape(1, FEAT))
        return i + 1, m_next, h

    _, _, h_fin = lax.while_loop(
        cond, body, (0, m0, jnp.zeros((NODES, FEAT), jnp.float32)))
    return _tc_read(h_fin, first_nodes_idx.reshape(BATCH, 1))


# speedup vs baseline: 7.0774x; 3.8416x over previous
import functools

import jax
import jax.numpy as jnp
from jax import lax
from jax.experimental import pallas as pl
from jax.experimental.pallas import tpu as pltpu
from jax.experimental.pallas import tpu_sc as plsc

NODES = 10000
EDGES = 160000
FEAT = 256
HALF = 128
BATCH = 8
LAYERS = 2

NC = 2
NS = 16
EPT = EDGES // NS
K = 80
NCHK = EPT // K
NPAIR = (NCHK - 1) // 2
WROWS = 624
LEFT = NODES - NS * WROWS
LEFT0 = NS * WROWS
WCOPY = (WROWS // K, WROWS % K)
NPAD = 10240
RED = NPAD // NS


def _zero_1d(ref, n):
    def body(i, _):
        ref[pl.ds(i * 16, 16)] = jnp.zeros((16,), ref.dtype)
        return None

    lax.fori_loop(0, n // 16, body, None)


def _zero_2d(ref, nrows, ncols):
    cpl = ncols // 16

    def body(i, _):
        ref[i // cpl, pl.ds((i % cpl) * 16, 16)] = jnp.zeros((16,), ref.dtype)
        return None

    lax.fori_loop(0, nrows * cpl, body, None)


@functools.lru_cache(maxsize=None)
def _sc_kernels():
    mesh = plsc.VectorSubcoreMesh(
        core_axis_name="c", subcore_axis_name="s",
        num_cores=NC, num_subcores=NS)

    @functools.partial(
        pl.kernel,
        out_type=jax.ShapeDtypeStruct((2 * NPAD,), jnp.float32),
        mesh=mesh,
        scratch_types=[
            pltpu.VMEM((NPAD,), jnp.int32),
            pltpu.VMEM((NPAD,), jnp.float32),
            pltpu.VMEM((RED,), jnp.float32),
            pltpu.VMEM((RED,), jnp.float32),
            pltpu.VMEM_SHARED((NS, 1, NPAD), jnp.float32),
        ],
        compiler_params=pltpu.CompilerParams(needs_layout_passes=False),
    )
    def deg_kernel(edge_ref, out_ref, idx_full, hist_v, tmp_v, red_v, spm):
        c = lax.axis_index("c")
        s = lax.axis_index("s")

        _zero_1d(hist_v, NPAD)
        ones16 = jnp.ones((16,), jnp.float32)
        pltpu.sync_copy(edge_ref.at[pl.ds(c * EDGES + s * EPT, EPT)],
                        idx_full.at[pl.ds(0, EPT)])

        def padfill(j, _):
            idx_full[pl.ds(EPT + j * 16, 16)] = jnp.full(
                (16,), NODES, jnp.int32)
            return None

        lax.fori_loop(0, (NPAD - EPT) // 16, padfill, None)

        def upd(j, _):
            idx16 = idx_full[pl.ds(j * 16, 16)]
            plsc.addupdate_scatter(hist_v, [idx16], ones16)
            return None

        lax.fori_loop(0, NPAD // 16, upd, None)
        pltpu.sync_copy(hist_v, spm.at[s, 0])
        plsc.subcore_barrier()

        _zero_1d(red_v, RED)
        for t in range(NS):
            pltpu.sync_copy(spm.at[t, 0, pl.ds(s * RED, RED)], tmp_v)

            def acc(j, _):
                sl = pl.ds(j * 16, 16)
                red_v[sl] = red_v[sl] + tmp_v[sl]
                return None

            lax.fori_loop(0, RED // 16, acc, None)
        pltpu.sync_copy(red_v, out_ref.at[pl.ds(c * NPAD + s * RED, RED)])

    @functools.partial(
        pl.kernel,
        out_type=jax.ShapeDtypeStruct((2 * NODES, HALF), jnp.float32),
        mesh=mesh,
        scratch_types=[
            pltpu.VMEM((EPT,), jnp.int32),
            pltpu.VMEM((EPT,), jnp.int32),
            pltpu.VMEM((K,), jnp.int32),
            pltpu.VMEM((K,), jnp.int32),
            pltpu.VMEM((K,), jnp.int32),
            pltpu.VMEM((K,), jnp.int32),
            pltpu.VMEM((K, HALF), jnp.float32),
            pltpu.VMEM((K, HALF), jnp.float32),
            pltpu.VMEM_SHARED((NODES, HALF), jnp.float32),
            pltpu.SemaphoreType.DMA,
            pltpu.SemaphoreType.DMA,
        ],
    )
    def scatter_kernel(m_ref, edge_ref, out_ref, src_full, dst_full,
                       src_va, dst_va, src_vb, dst_vb, rows_a, rows_b,
                       acc, sem_a, sem_b):
        c = lax.axis_index("c")
        s = lax.axis_index("s")

        pltpu.sync_copy(edge_ref.at[pl.ds(s * EPT, EPT)],
                        src_full.at[pl.ds(0, EPT)])
        pltpu.sync_copy(edge_ref.at[pl.ds(EDGES + s * EPT, EPT)],
                        dst_full.at[pl.ds(0, EPT)])

        def soff(j, _):
            sl = pl.ds(j * 16, 16)
            src_full[sl] = src_full[sl] + c * NODES
            return None

        lax.fori_loop(0, EPT // 16, soff, None)

        _zero_2d(rows_a, K, HALF)
        for i in range(WCOPY[0]):
            pltpu.sync_copy(rows_a, acc.at[pl.ds(s * WROWS + i * K, K)])
        pltpu.sync_copy(rows_a.at[pl.ds(0, WCOPY[1])],
                        acc.at[pl.ds(s * WROWS + WCOPY[0] * K, WCOPY[1])])

        @pl.when(s == 0)
        def _():
            pltpu.sync_copy(rows_a.at[pl.ds(0, LEFT)],
                            acc.at[pl.ds(LEFT0, LEFT)])

        plsc.subcore_barrier()

        def fill_idx(chunk, sbuf, dbuf):
            def f(j, _):
                sl = pl.ds(j * 16, 16)
                off = pl.ds(chunk * K + j * 16, 16)
                sbuf[sl] = src_full[off]
                dbuf[sl] = dst_full[off]
                return None

            lax.fori_loop(0, K // 16, f, None)

        fill_idx(0, src_va, dst_va)
        pltpu.async_copy(m_ref.at[src_va], rows_a, sem_a)

        def pair(kk, _):
            fill_idx(2 * kk + 1, src_vb, dst_vb)
            pltpu.async_copy(m_ref.at[src_vb], rows_b, sem_b)
            pltpu.make_async_copy(m_ref.at[src_va], rows_a, sem_a).wait()
            pltpu.sync_copy(rows_a, acc.at[dst_va], add=True)

            @pl.when(kk < NPAIR - 1)
            def _():
                fill_idx(2 * kk + 2, src_va, dst_va)
                pltpu.async_copy(m_ref.at[src_va], rows_a, sem_a)

            pltpu.make_async_copy(m_ref.at[src_vb], rows_b, sem_b).wait()
            pltpu.sync_copy(rows_b, acc.at[dst_vb], add=True)
            return None

        lax.fori_loop(0, NPAIR, pair, None)
        fill_idx(NCHK - 1, src_va, dst_va)
        pltpu.async_copy(m_ref.at[src_va], rows_a, sem_a).wait()
        pltpu.sync_copy(rows_a, acc.at[dst_va], add=True)

        plsc.subcore_barrier()
        for i in range(WCOPY[0]):
            sl_a = pl.ds(s * WROWS + i * K, K)
            sl_o = pl.ds(c * NODES + s * WROWS + i * K, K)
            pltpu.sync_copy(acc.at[sl_a], rows_a)
            pltpu.sync_copy(rows_a, out_ref.at[sl_o])
        pltpu.sync_copy(acc.at[pl.ds(s * WROWS + WCOPY[0] * K, WCOPY[1])],
                        rows_a.at[pl.ds(0, WCOPY[1])])
        pltpu.sync_copy(rows_a.at[pl.ds(0, WCOPY[1])],
                        out_ref.at[pl.ds(c * NODES + s * WROWS + WCOPY[0] * K,
                                         WCOPY[1])])

        @pl.when(s == 0)
        def _():
            pltpu.sync_copy(acc.at[pl.ds(LEFT0, LEFT)],
                            rows_a.at[pl.ds(0, LEFT)])
            pltpu.sync_copy(rows_a.at[pl.ds(0, LEFT)],
                            out_ref.at[pl.ds(c * NODES + LEFT0, LEFT)])

    return deg_kernel, scatter_kernel


R = 1000
GRID = NODES // R


def _ln(xb, g, b):
    mu = jnp.mean(xb, axis=-1, keepdims=True)
    xc = xb - mu
    var = jnp.mean(xc * xc, axis=-1, keepdims=True)
    return xc * lax.rsqrt(var + 1e-5) * g + b


def _inv_sqrt_deg(deg_blk):
    return lax.rsqrt(jnp.maximum(deg_blk, 1.0))


_VEC_SPEC = pl.BlockSpec((1, FEAT), lambda i: (0, 0))
_DEG_SPEC = pl.BlockSpec((R, 1), lambda i: (i, 0))
_MH_SPEC = pl.BlockSpec((2, R, HALF), lambda i: (0, i, 0))
_MH_SHAPE = jax.ShapeDtypeStruct((2, NODES, HALF), jnp.float32)


def _store_halves(m, m_ref):
    m_ref[0] = m[:, :HALF]
    m_ref[1] = m[:, HALF:]


def _tc_pre_body(x_ref, od_ref, g1_ref, b1_ref, m_ref):
    hn = _ln(x_ref[...], g1_ref[...], b1_ref[...])
    _store_halves(hn * _inv_sqrt_deg(od_ref[...]), m_ref)


_tc_pre = pl.pallas_call(
    _tc_pre_body,
    grid=(GRID,),
    in_specs=[
        pl.BlockSpec((R, FEAT), lambda i: (i, 0)),
        _DEG_SPEC,
        _VEC_SPEC,
        _VEC_SPEC,
    ],
    out_specs=_MH_SPEC,
    out_shape=_MH_SHAPE,
)


def _tc_mid_body(agg_ref, ind_ref, od_ref, wg_ref, bg_ref, g2_ref, b2_ref,
                 w1_ref, b1f_ref, w2_ref, b2f_ref, g1n_ref, b1n_ref,
                 m_ref, h_ref):
    a = jnp.concatenate([agg_ref[0], agg_ref[1]], axis=-1)
    a = a * _inv_sqrt_deg(ind_ref[...])
    h = jnp.dot(a, wg_ref[...], preferred_element_type=jnp.float32)
    h = h + bg_ref[...]
    hn2 = _ln(h, g2_ref[...], b2_ref[...])
    t = jnp.dot(hn2, w1_ref[...], preferred_element_type=jnp.float32)
    t = jnp.maximum(t + b1f_ref[...], 0.0)
    ff = jnp.dot(t, w2_ref[...], preferred_element_type=jnp.float32)
    h = ff + b2f_ref[...] + h
    h_ref[...] = h
    hn = _ln(h, g1n_ref[...], b1n_ref[...])
    _store_halves(hn * _inv_sqrt_deg(od_ref[...]), m_ref)


_tc_mid = pl.pallas_call(
    _tc_mid_body,
    grid=(GRID,),
    in_specs=[
        _MH_SPEC,
        _DEG_SPEC,
        _DEG_SPEC,
        pl.BlockSpec((FEAT, FEAT), lambda i: (0, 0)),
        _VEC_SPEC,
        _VEC_SPEC,
        _VEC_SPEC,
        pl.BlockSpec((FEAT, 2 * FEAT), lambda i: (0, 0)),
        pl.BlockSpec((1, 2 * FEAT), lambda i: (0, 0)),
        pl.BlockSpec((2 * FEAT, FEAT), lambda i: (0, 0)),
        _VEC_SPEC,
        _VEC_SPEC,
        _VEC_SPEC,
    ],
    out_specs=[_MH_SPEC, pl.BlockSpec((R, FEAT), lambda i: (i, 0))],
    out_shape=[_MH_SHAPE, jax.ShapeDtypeStruct((NODES, FEAT), jnp.float32)],
)


def _tc_read_body(h_ref, fni_ref, out_ref):
    i = pl.program_id(0)
    rows = lax.broadcasted_iota(jnp.int32, (BATCH, R), 1) + i * R
    onehot = (rows == fni_ref[...]).astype(jnp.float32)
    contrib = jnp.dot(onehot, h_ref[...], preferred_element_type=jnp.float32)

    @pl.when(i == 0)
    def _():
        out_ref[...] = contrib

    @pl.when(i > 0)
    def _():
        out_ref[...] += contrib


_tc_read = pl.pallas_call(
    _tc_read_body,
    grid=(GRID,),
    in_specs=[
        pl.BlockSpec((R, FEAT), lambda i: (i, 0)),
        pl.BlockSpec((BATCH, 1), lambda i: (0, 0)),
    ],
    out_specs=pl.BlockSpec((BATCH, FEAT), lambda i: (0, 0)),
    out_shape=jax.ShapeDtypeStruct((BATCH, FEAT), jnp.float32),
)


def kernel(x, edge_index, first_nodes_idx, ln1_g, ln1_b, Wg, bg, ln2_g,
           ln2_b, W1, b1, W2, b2):
    deg_kernel, scatter_kernel = _sc_kernels()
    edge_flat = edge_index.reshape(2 * EDGES)
    deg = deg_kernel(edge_flat)
    od = deg[:NODES].reshape(NODES, 1)
    ind = deg[NPAD:NPAD + NODES].reshape(NODES, 1)

    m0 = _tc_pre(x, od, ln1_g[0:1], ln1_b[0:1])

    ln1n_g = jnp.roll(ln1_g, -1, axis=0)
    ln1n_b = jnp.roll(ln1_b, -1, axis=0)

    limit = LAYERS + (edge_flat[0] >> 31)

    def cond(st):
        return st[0] < limit

    def body(st):
        i, m, _ = st
        agg = scatter_kernel(m.reshape(2 * NODES, HALF), edge_flat)
        m_next, h = _tc_mid(
            agg.reshape(2, NODES, HALF), ind, od, Wg[i],
            bg[i].reshape(1, FEAT), ln2_g[i].reshape(1, FEAT),
            ln2_b[i].reshape(1, FEAT), W1[i], b1[i].reshape(1, 2 * FEAT),
            W2[i], b2[i].reshape(1, FEAT), ln1n_g[i].reshape(1, FEAT),
            ln1n_b[i].reshape(1, FEAT))
        return i + 1, m_next, h

    _, _, h_fin = lax.while_loop(
        cond, body, (0, m0, jnp.zeros((NODES, FEAT), jnp.float32)))
    return _tc_read(h_fin, first_nodes_idx.reshape(BATCH, 1))


# speedup vs baseline: 7.6333x; 1.0785x over previous
import functools

import jax
import jax.numpy as jnp
from jax import lax
from jax.experimental import pallas as pl
from jax.experimental.pallas import tpu as pltpu
from jax.experimental.pallas import tpu_sc as plsc

NODES = 10000
EDGES = 160000
FEAT = 256
HALF = 128
BATCH = 8
LAYERS = 2

NC = 2
NS = 16
EPT = EDGES // NS
K = 80
NCHK = EPT // K
NPAIR = (NCHK - 1) // 2
WROWS = 624
LEFT = NODES - NS * WROWS
LEFT0 = NS * WROWS
WCOPY = (WROWS // K, WROWS % K)
NPAD = 10240
RED = NPAD // NS


def _zero_1d(ref, n):
    def body(i, _):
        ref[pl.ds(i * 16, 16)] = jnp.zeros((16,), ref.dtype)
        return None

    lax.fori_loop(0, n // 16, body, None)


def _zero_2d(ref, nrows, ncols):
    cpl = ncols // 16

    def body(i, _):
        ref[i // cpl, pl.ds((i % cpl) * 16, 16)] = jnp.zeros((16,), ref.dtype)
        return None

    lax.fori_loop(0, nrows * cpl, body, None)


@functools.lru_cache(maxsize=None)
def _sc_kernels():
    mesh = plsc.VectorSubcoreMesh(
        core_axis_name="c", subcore_axis_name="s",
        num_cores=NC, num_subcores=NS)

    @functools.partial(
        pl.kernel,
        out_type=jax.ShapeDtypeStruct((2 * NPAD,), jnp.float32),
        mesh=mesh,
        scratch_types=[
            pltpu.VMEM((NPAD,), jnp.int32),
            pltpu.VMEM((NPAD,), jnp.float32),
            pltpu.VMEM((RED,), jnp.float32),
            pltpu.VMEM((RED,), jnp.float32),
            pltpu.VMEM_SHARED((NS, 1, NPAD), jnp.float32),
        ],
        compiler_params=pltpu.CompilerParams(needs_layout_passes=False),
    )
    def deg_kernel(edge_ref, out_ref, idx_full, hist_v, tmp_v, red_v, spm):
        c = lax.axis_index("c")
        s = lax.axis_index("s")

        _zero_1d(hist_v, NPAD)
        ones16 = jnp.ones((16,), jnp.float32)
        pltpu.sync_copy(edge_ref.at[pl.ds(c * EDGES + s * EPT, EPT)],
                        idx_full.at[pl.ds(0, EPT)])

        def padfill(j, _):
            idx_full[pl.ds(EPT + j * 16, 16)] = jnp.full(
                (16,), NODES, jnp.int32)
            return None

        lax.fori_loop(0, (NPAD - EPT) // 16, padfill, None)

        def upd(j, _):
            idx16 = idx_full[pl.ds(j * 16, 16)]
            plsc.addupdate_scatter(hist_v, [idx16], ones16)
            return None

        lax.fori_loop(0, NPAD // 16, upd, None)
        pltpu.sync_copy(hist_v, spm.at[s, 0])
        plsc.subcore_barrier()

        _zero_1d(red_v, RED)
        for t in range(NS):
            pltpu.sync_copy(spm.at[t, 0, pl.ds(s * RED, RED)], tmp_v)

            def acc(j, _):
                sl = pl.ds(j * 16, 16)
                red_v[sl] = red_v[sl] + tmp_v[sl]
                return None

            lax.fori_loop(0, RED // 16, acc, None)
        pltpu.sync_copy(red_v, out_ref.at[pl.ds(c * NPAD + s * RED, RED)])

    @functools.partial(
        pl.kernel,
        out_type=jax.ShapeDtypeStruct((2 * NODES, HALF), jnp.float32),
        mesh=mesh,
        scratch_types=[
            pltpu.VMEM((EPT,), jnp.int32),
            pltpu.VMEM((EPT,), jnp.int32),
            pltpu.VMEM((K,), jnp.int32),
            pltpu.VMEM((K,), jnp.int32),
            pltpu.VMEM((K,), jnp.int32),
            pltpu.VMEM((K,), jnp.int32),
            pltpu.VMEM((K, HALF), jnp.float32),
            pltpu.VMEM((K, HALF), jnp.float32),
            pltpu.VMEM_SHARED((NODES, HALF), jnp.float32),
            pltpu.SemaphoreType.DMA,
            pltpu.SemaphoreType.DMA,
        ],
    )
    def scatter_kernel(m_ref, edge_ref, out_ref, src_full, dst_full,
                       src_va, dst_va, src_vb, dst_vb, rows_a, rows_b,
                       acc, sem_a, sem_b):
        c = lax.axis_index("c")
        s = lax.axis_index("s")

        pltpu.sync_copy(edge_ref.at[pl.ds(s * EPT, EPT)],
                        src_full.at[pl.ds(0, EPT)])
        pltpu.sync_copy(edge_ref.at[pl.ds(EDGES + s * EPT, EPT)],
                        dst_full.at[pl.ds(0, EPT)])

        def soff(j, _):
            sl = pl.ds(j * 16, 16)
            src_full[sl] = src_full[sl] + c * NODES
            return None

        lax.fori_loop(0, EPT // 16, soff, None)

        _zero_2d(rows_a, K, HALF)
        for i in range(WCOPY[0]):
            pltpu.sync_copy(rows_a, acc.at[pl.ds(s * WROWS + i * K, K)])
        pltpu.sync_copy(rows_a.at[pl.ds(0, WCOPY[1])],
                        acc.at[pl.ds(s * WROWS + WCOPY[0] * K, WCOPY[1])])

        @pl.when(s == 0)
        def _():
            pltpu.sync_copy(rows_a.at[pl.ds(0, LEFT)],
                            acc.at[pl.ds(LEFT0, LEFT)])

        plsc.subcore_barrier()

        def fill_idx(chunk, sbuf, dbuf):
            def f(j, _):
                sl = pl.ds(j * 16, 16)
                off = pl.ds(chunk * K + j * 16, 16)
                sbuf[sl] = src_full[off]
                dbuf[sl] = dst_full[off]
                return None

            lax.fori_loop(0, K // 16, f, None)

        fill_idx(0, src_va, dst_va)
        pltpu.async_copy(m_ref.at[src_va], rows_a, sem_a)

        def pair(kk, _):
            fill_idx(2 * kk + 1, src_vb, dst_vb)
            pltpu.async_copy(m_ref.at[src_vb], rows_b, sem_b)
            pltpu.make_async_copy(m_ref.at[src_va], rows_a, sem_a).wait()
            pltpu.sync_copy(rows_a, acc.at[dst_va], add=True)

            @pl.when(kk < NPAIR - 1)
            def _():
                fill_idx(2 * kk + 2, src_va, dst_va)
                pltpu.async_copy(m_ref.at[src_va], rows_a, sem_a)

            pltpu.make_async_copy(m_ref.at[src_vb], rows_b, sem_b).wait()
            pltpu.sync_copy(rows_b, acc.at[dst_vb], add=True)
            return None

        lax.fori_loop(0, NPAIR, pair, None)
        fill_idx(NCHK - 1, src_va, dst_va)
        pltpu.async_copy(m_ref.at[src_va], rows_a, sem_a).wait()
        pltpu.sync_copy(rows_a, acc.at[dst_va], add=True)

        plsc.subcore_barrier()
        for i in range(WCOPY[0]):
            sl_a = pl.ds(s * WROWS + i * K, K)
            sl_o = pl.ds(c * NODES + s * WROWS + i * K, K)
            pltpu.sync_copy(acc.at[sl_a], rows_a)
            pltpu.sync_copy(rows_a, out_ref.at[sl_o])
        pltpu.sync_copy(acc.at[pl.ds(s * WROWS + WCOPY[0] * K, WCOPY[1])],
                        rows_a.at[pl.ds(0, WCOPY[1])])
        pltpu.sync_copy(rows_a.at[pl.ds(0, WCOPY[1])],
                        out_ref.at[pl.ds(c * NODES + s * WROWS + WCOPY[0] * K,
                                         WCOPY[1])])

        @pl.when(s == 0)
        def _():
            pltpu.sync_copy(acc.at[pl.ds(LEFT0, LEFT)],
                            rows_a.at[pl.ds(0, LEFT)])
            pltpu.sync_copy(rows_a.at[pl.ds(0, LEFT)],
                            out_ref.at[pl.ds(c * NODES + LEFT0, LEFT)])

    return deg_kernel, scatter_kernel


R = 1000
GRID = NODES // R


def _ln(xb, g, b):
    mu = jnp.mean(xb, axis=-1, keepdims=True)
    xc = xb - mu
    var = jnp.mean(xc * xc, axis=-1, keepdims=True)
    return xc * lax.rsqrt(var + 1e-5) * g + b


def _inv_sqrt_deg(deg_blk):
    return lax.rsqrt(jnp.maximum(deg_blk, 1.0))


_VEC_SPEC = pl.BlockSpec((1, FEAT), lambda i: (0, 0))
_DEG_SPEC = pl.BlockSpec((R, 1), lambda i: (i, 0))
_MH_SPEC = pl.BlockSpec((2, R, HALF), lambda i: (0, i, 0))
_MH_SHAPE = jax.ShapeDtypeStruct((2, NODES, HALF), jnp.float32)


def _store_halves(m, m_ref):
    m_ref[0] = m[:, :HALF]
    m_ref[1] = m[:, HALF:]


def _tc_pre_body(x_ref, od_ref, g1_ref, b1_ref, m_ref):
    hn = _ln(x_ref[...], g1_ref[...], b1_ref[...])
    _store_halves(hn * _inv_sqrt_deg(od_ref[...]), m_ref)


_tc_pre = pl.pallas_call(
    _tc_pre_body,
    grid=(GRID,),
    in_specs=[
        pl.BlockSpec((R, FEAT), lambda i: (i, 0)),
        _DEG_SPEC,
        _VEC_SPEC,
        _VEC_SPEC,
    ],
    out_specs=_MH_SPEC,
    out_shape=_MH_SHAPE,
)


def _dense_layer(agg_ref, ind_ref, wg_ref, bg_ref, g2_ref, b2_ref,
                 w1_ref, b1f_ref, w2_ref, b2f_ref):
    a = jnp.concatenate([agg_ref[0], agg_ref[1]], axis=-1)
    a = a * _inv_sqrt_deg(ind_ref[...])
    h = jnp.dot(a, wg_ref[...], preferred_element_type=jnp.float32)
    h = h + bg_ref[...]
    hn2 = _ln(h, g2_ref[...], b2_ref[...])
    t = jnp.dot(hn2, w1_ref[...], preferred_element_type=jnp.float32)
    t = jnp.maximum(t + b1f_ref[...], 0.0)
    ff = jnp.dot(t, w2_ref[...], preferred_element_type=jnp.float32)
    return ff + b2f_ref[...] + h


def _tc_mid_body(agg_ref, ind_ref, od_ref, wg_ref, bg_ref, g2_ref, b2_ref,
                 w1_ref, b1f_ref, w2_ref, b2f_ref, g1n_ref, b1n_ref, m_ref):
    h = _dense_layer(agg_ref, ind_ref, wg_ref, bg_ref, g2_ref, b2_ref,
                     w1_ref, b1f_ref, w2_ref, b2f_ref)
    hn = _ln(h, g1n_ref[...], b1n_ref[...])
    _store_halves(hn * _inv_sqrt_deg(od_ref[...]), m_ref)


_tc_mid = pl.pallas_call(
    _tc_mid_body,
    grid=(GRID,),
    in_specs=[
        _MH_SPEC,
        _DEG_SPEC,
        _DEG_SPEC,
        pl.BlockSpec((FEAT, FEAT), lambda i: (0, 0)),
        _VEC_SPEC,
        _VEC_SPEC,
        _VEC_SPEC,
        pl.BlockSpec((FEAT, 2 * FEAT), lambda i: (0, 0)),
        pl.BlockSpec((1, 2 * FEAT), lambda i: (0, 0)),
        pl.BlockSpec((2 * FEAT, FEAT), lambda i: (0, 0)),
        _VEC_SPEC,
        _VEC_SPEC,
        _VEC_SPEC,
    ],
    out_specs=_MH_SPEC,
    out_shape=_MH_SHAPE,
)


def _tc_fin_body(agg_ref, ind_ref, wg_ref, bg_ref, g2_ref, b2_ref,
                 w1_ref, b1f_ref, w2_ref, b2f_ref, fni_ref, out_ref):
    h = _dense_layer(agg_ref, ind_ref, wg_ref, bg_ref, g2_ref, b2_ref,
                     w1_ref, b1f_ref, w2_ref, b2f_ref)
    i = pl.program_id(0)
    rows = lax.broadcasted_iota(jnp.int32, (BATCH, R), 1) + i * R
    onehot = (rows == fni_ref[...]).astype(jnp.float32)
    contrib = jnp.dot(onehot, h, preferred_element_type=jnp.float32)

    @pl.when(i == 0)
    def _():
        out_ref[...] = contrib

    @pl.when(i > 0)
    def _():
        out_ref[...] += contrib


_W_SPECS = [
    pl.BlockSpec((FEAT, FEAT), lambda i: (0, 0)),
    _VEC_SPEC,
    _VEC_SPEC,
    _VEC_SPEC,
    pl.BlockSpec((FEAT, 2 * FEAT), lambda i: (0, 0)),
    pl.BlockSpec((1, 2 * FEAT), lambda i: (0, 0)),
    pl.BlockSpec((2 * FEAT, FEAT), lambda i: (0, 0)),
    _VEC_SPEC,
]

_tc_fin = pl.pallas_call(
    _tc_fin_body,
    grid=(GRID,),
    in_specs=[_MH_SPEC, _DEG_SPEC] + _W_SPECS
    + [pl.BlockSpec((BATCH, 1), lambda i: (0, 0))],
    out_specs=pl.BlockSpec((BATCH, FEAT), lambda i: (0, 0)),
    out_shape=jax.ShapeDtypeStruct((BATCH, FEAT), jnp.float32),
)


def kernel(x, edge_index, first_nodes_idx, ln1_g, ln1_b, Wg, bg, ln2_g,
           ln2_b, W1, b1, W2, b2):
    deg_kernel, scatter_kernel = _sc_kernels()
    edge_flat = edge_index.reshape(2 * EDGES)
    deg = deg_kernel(edge_flat)
    od = deg[:NODES].reshape(NODES, 1)
    ind = deg[NPAD:NPAD + NODES].reshape(NODES, 1)

    m = _tc_pre(x, od, ln1_g[0:1], ln1_b[0:1])
    agg = scatter_kernel(m.reshape(2 * NODES, HALF), edge_flat)
    m = _tc_mid(agg.reshape(2, NODES, HALF), ind, od, Wg[0],
                bg[0:1], ln2_g[0:1], ln2_b[0:1], W1[0], b1[0:1],
                W2[0], b2[0:1], ln1_g[1:2], ln1_b[1:2])
    agg = scatter_kernel(m.reshape(2 * NODES, HALF), edge_flat)
    return _tc_fin(agg.reshape(2, NODES, HALF), ind, Wg[1],
                   bg[1:2], ln2_g[1:2], ln2_b[1:2], W1[1], b1[1:2],
                   W2[1], b2[1:2], first_nodes_idx.reshape(BATCH, 1))


# speedup vs baseline: 8.7372x; 1.1446x over previous
import functools

import jax
import jax.numpy as jnp
from jax import lax
from jax.experimental import pallas as pl
from jax.experimental.pallas import tpu as pltpu
from jax.experimental.pallas import tpu_sc as plsc

NODES = 10000
EDGES = 160000
FEAT = 256
HALF = 128
BATCH = 8
LAYERS = 2

NC = 2
NS = 16
EPT = EDGES // NS
K = 80
NCHK = EPT // K
NPAIR = (NCHK - 1) // 2
WROWS = 624
LEFT = NODES - NS * WROWS
LEFT0 = NS * WROWS
WCOPY = (WROWS // K, WROWS % K)
NPAD = 10240
RED = NPAD // NS


def _zero_1d(ref, n):
    def body(i, _):
        ref[pl.ds(i * 16, 16)] = jnp.zeros((16,), ref.dtype)
        return None

    lax.fori_loop(0, n // 16, body, None)


def _zero_2d(ref, nrows, ncols):
    cpl = ncols // 16

    def body(i, _):
        ref[i // cpl, pl.ds((i % cpl) * 16, 16)] = jnp.zeros((16,), ref.dtype)
        return None

    lax.fori_loop(0, nrows * cpl, body, None)


@functools.lru_cache(maxsize=None)
def _sc_kernels():
    mesh = plsc.VectorSubcoreMesh(
        core_axis_name="c", subcore_axis_name="s",
        num_cores=NC, num_subcores=NS)

    @functools.partial(
        pl.kernel,
        out_type=jax.ShapeDtypeStruct((2 * NPAD,), jnp.float32),
        mesh=mesh,
        scratch_types=[
            pltpu.VMEM((NPAD,), jnp.int32),
            pltpu.VMEM((NPAD,), jnp.float32),
            pltpu.VMEM((RED,), jnp.float32),
            pltpu.VMEM((RED,), jnp.float32),
            pltpu.VMEM_SHARED((NS, 1, NPAD), jnp.float32),
        ],
        compiler_params=pltpu.CompilerParams(needs_layout_passes=False),
    )
    def deg_kernel(edge_ref, out_ref, idx_full, hist_v, tmp_v, red_v, spm):
        c = lax.axis_index("c")
        s = lax.axis_index("s")

        _zero_1d(hist_v, NPAD)
        ones16 = jnp.ones((16,), jnp.float32)
        pltpu.sync_copy(edge_ref.at[pl.ds(c * EDGES + s * EPT, EPT)],
                        idx_full.at[pl.ds(0, EPT)])

        def padfill(j, _):
            idx_full[pl.ds(EPT + j * 16, 16)] = jnp.full(
                (16,), NODES, jnp.int32)
            return None

        lax.fori_loop(0, (NPAD - EPT) // 16, padfill, None)

        def upd(j, _):
            idx16 = idx_full[pl.ds(j * 16, 16)]
            plsc.addupdate_scatter(hist_v, [idx16], ones16)
            return None

        lax.fori_loop(0, NPAD // 16, upd, None)
        pltpu.sync_copy(hist_v, spm.at[s, 0])
        plsc.subcore_barrier()

        _zero_1d(red_v, RED)
        for t in range(NS):
            pltpu.sync_copy(spm.at[t, 0, pl.ds(s * RED, RED)], tmp_v)

            def acc(j, _):
                sl = pl.ds(j * 16, 16)
                red_v[sl] = red_v[sl] + tmp_v[sl]
                return None

            lax.fori_loop(0, RED // 16, acc, None)
        pltpu.sync_copy(red_v, out_ref.at[pl.ds(c * NPAD + s * RED, RED)])

    @functools.partial(
        pl.kernel,
        out_type=jax.ShapeDtypeStruct((2 * NODES, HALF), jnp.float32),
        mesh=mesh,
        scratch_types=[
            pltpu.VMEM((EPT,), jnp.int32),
            pltpu.VMEM((K,), jnp.int32),
            pltpu.VMEM((K,), jnp.int32),
            pltpu.VMEM((K,), jnp.int32),
            pltpu.VMEM((K, HALF), jnp.float32),
            pltpu.VMEM((K, HALF), jnp.float32),
            pltpu.VMEM((K, HALF), jnp.float32),
            pltpu.VMEM_SHARED((NODES, HALF), jnp.float32),
            pltpu.SemaphoreType.DMA,
            pltpu.SemaphoreType.DMA,
            pltpu.SemaphoreType.DMA,
            pltpu.SemaphoreType.DMA,
            pltpu.SemaphoreType.DMA,
            pltpu.SemaphoreType.DMA,
            pltpu.SemaphoreType.DMA,
            pltpu.SemaphoreType.DMA,
            pltpu.SemaphoreType.DMA,
        ],
    )
    def scatter_kernel(m_ref, edge_ref, out_ref, src_full,
                       dst_v0, dst_v1, dst_v2, rows_0, rows_1, rows_2,
                       acc, gs_0, gs_1, gs_2, ss_0, ss_1, ss_2,
                       ds_0, ds_1, ds_2):
        c = lax.axis_index("c")
        s = lax.axis_index("s")

        pltpu.sync_copy(edge_ref.at[pl.ds(s * EPT, EPT)],
                        src_full.at[pl.ds(0, EPT)])

        def soff(j, _):
            sl = pl.ds(j * 16, 16)
            src_full[sl] = src_full[sl] + c * NODES
            return None

        lax.fori_loop(0, EPT // 16, soff, None)

        _zero_2d(rows_0, K, HALF)
        for i in range(WCOPY[0]):
            pltpu.sync_copy(rows_0, acc.at[pl.ds(s * WROWS + i * K, K)])
        pltpu.sync_copy(rows_0.at[pl.ds(0, WCOPY[1])],
                        acc.at[pl.ds(s * WROWS + WCOPY[0] * K, WCOPY[1])])

        @pl.when(s == 0)
        def _():
            pltpu.sync_copy(rows_0.at[pl.ds(0, LEFT)],
                            acc.at[pl.ds(LEFT0, LEFT)])

        plsc.subcore_barrier()

        rows = (rows_0, rows_1, rows_2)
        dsts = (dst_v0, dst_v1, dst_v2)
        gsem = (gs_0, gs_1, gs_2)
        ssem = (ss_0, ss_1, ss_2)
        dsem = (ds_0, ds_1, ds_2)

        def fire_gather(chunk, u):
            pltpu.async_copy(m_ref.at[src_full.at[pl.ds(chunk * K, K)]],
                             rows[u], gsem[u])

        def wait_gather(chunk, u):
            pltpu.make_async_copy(m_ref.at[src_full.at[pl.ds(chunk * K, K)]],
                                  rows[u], gsem[u]).wait()

        def _dst_slice(chunk):
            return edge_ref.at[pl.ds(EDGES + s * EPT + chunk * K, K)]

        def fire_dst(chunk, u):
            pltpu.async_copy(_dst_slice(chunk), dsts[u], dsem[u])

        def wait_dst(chunk, u):
            pltpu.make_async_copy(_dst_slice(chunk), dsts[u], dsem[u]).wait()

        def fire_scatter(u):
            pltpu.async_copy(rows[u], acc.at[dsts[u]], ssem[u], add=True)

        def wait_scatter(u):
            pltpu.make_async_copy(rows[u], acc.at[dsts[u]], ssem[u]).wait()

        for u in (0, 1):
            fire_dst(u, u)
            fire_gather(u, u)

        def triple(kk, _):
            for u in range(3):
                c = 3 * kk + u
                wait_gather(c, u)
                wait_dst(c, u)
                fire_scatter(u)
                z = (u + 2) % 3

                @pl.when(c >= 1)
                def _():
                    wait_scatter(z)

                fire_dst(c + 2, z)
                fire_gather(c + 2, z)
            return None

        lax.fori_loop(0, (NCHK - 2) // 3, triple, None)
        for tail_c, u in ((NCHK - 2, 0), (NCHK - 1, 1)):
            wait_gather(tail_c, u)
            wait_dst(tail_c, u)
            fire_scatter(u)
        wait_scatter(2)
        wait_scatter(0)
        wait_scatter(1)

        plsc.subcore_barrier()
        for i in range(WCOPY[0]):
            sl_a = pl.ds(s * WROWS + i * K, K)
            sl_o = pl.ds(c * NODES + s * WROWS + i * K, K)
            pltpu.sync_copy(acc.at[sl_a], rows_0)
            pltpu.sync_copy(rows_0, out_ref.at[sl_o])
        pltpu.sync_copy(acc.at[pl.ds(s * WROWS + WCOPY[0] * K, WCOPY[1])],
                        rows_0.at[pl.ds(0, WCOPY[1])])
        pltpu.sync_copy(rows_0.at[pl.ds(0, WCOPY[1])],
                        out_ref.at[pl.ds(c * NODES + s * WROWS + WCOPY[0] * K,
                                         WCOPY[1])])

        @pl.when(s == 0)
        def _():
            pltpu.sync_copy(acc.at[pl.ds(LEFT0, LEFT)],
                            rows_0.at[pl.ds(0, LEFT)])
            pltpu.sync_copy(rows_0.at[pl.ds(0, LEFT)],
                            out_ref.at[pl.ds(c * NODES + LEFT0, LEFT)])

    return deg_kernel, scatter_kernel


R = 1000
GRID = NODES // R


def _ln(xb, g, b):
    mu = jnp.mean(xb, axis=-1, keepdims=True)
    xc = xb - mu
    var = jnp.mean(xc * xc, axis=-1, keepdims=True)
    return xc * lax.rsqrt(var + 1e-5) * g + b


def _inv_sqrt_deg(deg_blk):
    return lax.rsqrt(jnp.maximum(deg_blk, 1.0))


_VEC_SPEC = pl.BlockSpec((1, FEAT), lambda i: (0, 0))
_DEG_SPEC = pl.BlockSpec((R, 1), lambda i: (i, 0))
_MH_SPEC = pl.BlockSpec((2, R, HALF), lambda i: (0, i, 0))
_MH_SHAPE = jax.ShapeDtypeStruct((2, NODES, HALF), jnp.float32)


def _store_halves(m, m_ref):
    m_ref[0] = m[:, :HALF]
    m_ref[1] = m[:, HALF:]


def _tc_pre_body(x_ref, od_ref, g1_ref, b1_ref, m_ref):
    hn = _ln(x_ref[...], g1_ref[...], b1_ref[...])
    _store_halves(hn * _inv_sqrt_deg(od_ref[...]), m_ref)


_tc_pre = pl.pallas_call(
    _tc_pre_body,
    grid=(GRID,),
    in_specs=[
        pl.BlockSpec((R, FEAT), lambda i: (i, 0)),
        _DEG_SPEC,
        _VEC_SPEC,
        _VEC_SPEC,
    ],
    out_specs=_MH_SPEC,
    out_shape=_MH_SHAPE,
)


def _dense_layer(agg_ref, ind_ref, wg_ref, bg_ref, g2_ref, b2_ref,
                 w1_ref, b1f_ref, w2_ref, b2f_ref):
    a = jnp.concatenate([agg_ref[0], agg_ref[1]], axis=-1)
    a = a * _inv_sqrt_deg(ind_ref[...])
    h = jnp.dot(a, wg_ref[...], preferred_element_type=jnp.float32)
    h = h + bg_ref[...]
    hn2 = _ln(h, g2_ref[...], b2_ref[...])
    t = jnp.dot(hn2, w1_ref[...], preferred_element_type=jnp.float32)
    t = jnp.maximum(t + b1f_ref[...], 0.0)
    ff = jnp.dot(t, w2_ref[...], preferred_element_type=jnp.float32)
    return ff + b2f_ref[...] + h


def _tc_mid_body(agg_ref, ind_ref, od_ref, wg_ref, bg_ref, g2_ref, b2_ref,
                 w1_ref, b1f_ref, w2_ref, b2f_ref, g1n_ref, b1n_ref, m_ref):
    h = _dense_layer(agg_ref, ind_ref, wg_ref, bg_ref, g2_ref, b2_ref,
                     w1_ref, b1f_ref, w2_ref, b2f_ref)
    hn = _ln(h, g1n_ref[...], b1n_ref[...])
    _store_halves(hn * _inv_sqrt_deg(od_ref[...]), m_ref)


_tc_mid = pl.pallas_call(
    _tc_mid_body,
    grid=(GRID,),
    in_specs=[
        _MH_SPEC,
        _DEG_SPEC,
        _DEG_SPEC,
        pl.BlockSpec((FEAT, FEAT), lambda i: (0, 0)),
        _VEC_SPEC,
        _VEC_SPEC,
        _VEC_SPEC,
        pl.BlockSpec((FEAT, 2 * FEAT), lambda i: (0, 0)),
        pl.BlockSpec((1, 2 * FEAT), lambda i: (0, 0)),
        pl.BlockSpec((2 * FEAT, FEAT), lambda i: (0, 0)),
        _VEC_SPEC,
        _VEC_SPEC,
        _VEC_SPEC,
    ],
    out_specs=_MH_SPEC,
    out_shape=_MH_SHAPE,
)


def _tc_fin_body(agg_ref, ind_ref, wg_ref, bg_ref, g2_ref, b2_ref,
                 w1_ref, b1f_ref, w2_ref, b2f_ref, fni_ref, out_ref):
    h = _dense_layer(agg_ref, ind_ref, wg_ref, bg_ref, g2_ref, b2_ref,
                     w1_ref, b1f_ref, w2_ref, b2f_ref)
    i = pl.program_id(0)
    rows = lax.broadcasted_iota(jnp.int32, (BATCH, R), 1) + i * R
    onehot = (rows == fni_ref[...]).astype(jnp.float32)
    contrib = jnp.dot(onehot, h, preferred_element_type=jnp.float32)

    @pl.when(i == 0)
    def _():
        out_ref[...] = contrib

    @pl.when(i > 0)
    def _():
        out_ref[...] += contrib


_W_SPECS = [
    pl.BlockSpec((FEAT, FEAT), lambda i: (0, 0)),
    _VEC_SPEC,
    _VEC_SPEC,
    _VEC_SPEC,
    pl.BlockSpec((FEAT, 2 * FEAT), lambda i: (0, 0)),
    pl.BlockSpec((1, 2 * FEAT), lambda i: (0, 0)),
    pl.BlockSpec((2 * FEAT, FEAT), lambda i: (0, 0)),
    _VEC_SPEC,
]

_tc_fin = pl.pallas_call(
    _tc_fin_body,
    grid=(GRID,),
    in_specs=[_MH_SPEC, _DEG_SPEC] + _W_SPECS
    + [pl.BlockSpec((BATCH, 1), lambda i: (0, 0))],
    out_specs=pl.BlockSpec((BATCH, FEAT), lambda i: (0, 0)),
    out_shape=jax.ShapeDtypeStruct((BATCH, FEAT), jnp.float32),
)


def kernel(x, edge_index, first_nodes_idx, ln1_g, ln1_b, Wg, bg, ln2_g,
           ln2_b, W1, b1, W2, b2):
    deg_kernel, scatter_kernel = _sc_kernels()
    edge_flat = edge_index.reshape(2 * EDGES)
    deg = deg_kernel(edge_flat)
    od = deg[:NODES].reshape(NODES, 1)
    ind = deg[NPAD:NPAD + NODES].reshape(NODES, 1)

    m = _tc_pre(x, od, ln1_g[0:1], ln1_b[0:1])
    agg = scatter_kernel(m.reshape(2 * NODES, HALF), edge_flat)
    m = _tc_mid(agg.reshape(2, NODES, HALF), ind, od, Wg[0],
                bg[0:1], ln2_g[0:1], ln2_b[0:1], W1[0], b1[0:1],
                W2[0], b2[0:1], ln1_g[1:2], ln1_b[1:2])
    agg = scatter_kernel(m.reshape(2 * NODES, HALF), edge_flat)
    return _tc_fin(agg.reshape(2, NODES, HALF), ind, Wg[1],
                   bg[1:2], ln2_g[1:2], ln2_b[1:2], W1[1], b1[1:2],
                   W2[1], b2[1:2], first_nodes_idx.reshape(BATCH, 1))


# speedup vs baseline: 8.7632x; 1.0030x over previous
import functools

import jax
import jax.numpy as jnp
from jax import lax
from jax.experimental import pallas as pl
from jax.experimental.pallas import tpu as pltpu
from jax.experimental.pallas import tpu_sc as plsc

NODES = 10000
EDGES = 160000
FEAT = 256
HALF = 128
BATCH = 8
LAYERS = 2

NC = 2
NS = 16
EPT = EDGES // NS
K = 80
NCHK = EPT // K
NPAIR = (NCHK - 1) // 2
WROWS = 624
LEFT = NODES - NS * WROWS
LEFT0 = NS * WROWS
WCOPY = (WROWS // K, WROWS % K)
NPAD = 10240
RED = NPAD // NS


def _zero_1d(ref, n):
    def body(i, _):
        ref[pl.ds(i * 16, 16)] = jnp.zeros((16,), ref.dtype)
        return None

    lax.fori_loop(0, n // 16, body, None)


def _zero_2d(ref, nrows, ncols):
    cpl = ncols // 16

    def body(i, _):
        ref[i // cpl, pl.ds((i % cpl) * 16, 16)] = jnp.zeros((16,), ref.dtype)
        return None

    lax.fori_loop(0, nrows * cpl, body, None)


@functools.lru_cache(maxsize=None)
def _sc_kernels():
    mesh = plsc.VectorSubcoreMesh(
        core_axis_name="c", subcore_axis_name="s",
        num_cores=NC, num_subcores=NS)

    @functools.partial(
        pl.kernel,
        out_type=jax.ShapeDtypeStruct((2 * NPAD,), jnp.float32),
        mesh=mesh,
        scratch_types=[
            pltpu.VMEM((NPAD,), jnp.int32),
            pltpu.VMEM((NPAD,), jnp.float32),
            pltpu.VMEM((RED,), jnp.float32),
            pltpu.VMEM((RED,), jnp.float32),
            pltpu.VMEM_SHARED((NS, 1, NPAD), jnp.float32),
        ],
        compiler_params=pltpu.CompilerParams(needs_layout_passes=False),
    )
    def deg_kernel(edge_ref, out_ref, idx_full, hist_v, tmp_v, red_v, spm):
        c = lax.axis_index("c")
        s = lax.axis_index("s")

        _zero_1d(hist_v, NPAD)
        ones16 = jnp.ones((16,), jnp.float32)
        pltpu.sync_copy(edge_ref.at[pl.ds(c * EDGES + s * EPT, EPT)],
                        idx_full.at[pl.ds(0, EPT)])

        def padfill(j, _):
            idx_full[pl.ds(EPT + j * 16, 16)] = jnp.full(
                (16,), NODES, jnp.int32)
            return None

        lax.fori_loop(0, (NPAD - EPT) // 16, padfill, None)

        def upd(j, _):
            idx16 = idx_full[pl.ds(j * 16, 16)]
            plsc.addupdate_scatter(hist_v, [idx16], ones16)
            return None

        lax.fori_loop(0, NPAD // 16, upd, None)
        pltpu.sync_copy(hist_v, spm.at[s, 0])
        plsc.subcore_barrier()

        _zero_1d(red_v, RED)
        for t in range(NS):
            pltpu.sync_copy(spm.at[t, 0, pl.ds(s * RED, RED)], tmp_v)

            def acc(j, _):
                sl = pl.ds(j * 16, 16)
                red_v[sl] = red_v[sl] + tmp_v[sl]
                return None

            lax.fori_loop(0, RED // 16, acc, None)
        pltpu.sync_copy(red_v, out_ref.at[pl.ds(c * NPAD + s * RED, RED)])

    @functools.partial(
        pl.kernel,
        out_type=jax.ShapeDtypeStruct((2 * NODES, HALF), jnp.float32),
        mesh=mesh,
        scratch_types=[
            pltpu.VMEM((EPT,), jnp.int32),
            pltpu.VMEM((K,), jnp.int32),
            pltpu.VMEM((K,), jnp.int32),
            pltpu.VMEM((K,), jnp.int32),
            pltpu.VMEM((K, HALF), jnp.float32),
            pltpu.VMEM((K, HALF), jnp.float32),
            pltpu.VMEM((K, HALF), jnp.float32),
            pltpu.VMEM_SHARED((NODES, HALF), jnp.float32),
            pltpu.SemaphoreType.DMA,
            pltpu.SemaphoreType.DMA,
            pltpu.SemaphoreType.DMA,
            pltpu.SemaphoreType.DMA,
            pltpu.SemaphoreType.DMA,
            pltpu.SemaphoreType.DMA,
            pltpu.SemaphoreType.DMA,
            pltpu.SemaphoreType.DMA,
            pltpu.SemaphoreType.DMA,
        ],
    )
    def scatter_kernel(m_ref, edge_ref, out_ref, src_full,
                       dst_v0, dst_v1, dst_v2, rows_0, rows_1, rows_2,
                       acc, gs_0, gs_1, gs_2, ss_0, ss_1, ss_2,
                       ds_0, ds_1, ds_2):
        c = lax.axis_index("c")
        s = lax.axis_index("s")

        pltpu.sync_copy(edge_ref.at[pl.ds(s * EPT, EPT)],
                        src_full.at[pl.ds(0, EPT)])

        def soff(j, _):
            sl = pl.ds(j * 16, 16)
            src_full[sl] = src_full[sl] + c * NODES
            return None

        lax.fori_loop(0, EPT // 16, soff, None)

        _zero_2d(rows_0, K, HALF)
        for i in range(WCOPY[0]):
            pltpu.sync_copy(rows_0, acc.at[pl.ds(s * WROWS + i * K, K)])
        pltpu.sync_copy(rows_0.at[pl.ds(0, WCOPY[1])],
                        acc.at[pl.ds(s * WROWS + WCOPY[0] * K, WCOPY[1])])

        @pl.when(s == 0)
        def _():
            pltpu.sync_copy(rows_0.at[pl.ds(0, LEFT)],
                            acc.at[pl.ds(LEFT0, LEFT)])

        plsc.subcore_barrier()

        rows = (rows_0, rows_1, rows_2)
        dsts = (dst_v0, dst_v1, dst_v2)
        gsem = (gs_0, gs_1, gs_2)
        ssem = (ss_0, ss_1, ss_2)
        dsem = (ds_0, ds_1, ds_2)

        def fire_gather(chunk, u):
            pltpu.async_copy(m_ref.at[src_full.at[pl.ds(chunk * K, K)]],
                             rows[u], gsem[u])

        def wait_gather(chunk, u):
            pltpu.make_async_copy(m_ref.at[src_full.at[pl.ds(chunk * K, K)]],
                                  rows[u], gsem[u]).wait()

        def _dst_slice(chunk):
            return edge_ref.at[pl.ds(EDGES + s * EPT + chunk * K, K)]

        def fire_dst(chunk, u):
            pltpu.async_copy(_dst_slice(chunk), dsts[u], dsem[u])

        def wait_dst(chunk, u):
            pltpu.make_async_copy(_dst_slice(chunk), dsts[u], dsem[u]).wait()

        def fire_scatter(u):
            pltpu.async_copy(rows[u], acc.at[dsts[u]], ssem[u], add=True)

        def wait_scatter(u):
            pltpu.make_async_copy(rows[u], acc.at[dsts[u]], ssem[u]).wait()

        for u in (0, 1):
            fire_dst(u, u)
            fire_gather(u, u)

        def triple(kk, _):
            for u in range(3):
                c = 3 * kk + u
                wait_gather(c, u)
                wait_dst(c, u)
                fire_scatter(u)
                z = (u + 2) % 3

                @pl.when(c >= 1)
                def _():
                    wait_scatter(z)

                fire_dst(c + 2, z)
                fire_gather(c + 2, z)
            return None

        lax.fori_loop(0, (NCHK - 2) // 3, triple, None)
        for tail_c, u in ((NCHK - 2, 0), (NCHK - 1, 1)):
            wait_gather(tail_c, u)
            wait_dst(tail_c, u)
            fire_scatter(u)
        wait_scatter(2)
        wait_scatter(0)
        wait_scatter(1)

        plsc.subcore_barrier()
        for i in range(WCOPY[0]):
            sl_a = pl.ds(s * WROWS + i * K, K)
            sl_o = pl.ds(c * NODES + s * WROWS + i * K, K)
            pltpu.sync_copy(acc.at[sl_a], rows_0)
            pltpu.sync_copy(rows_0, out_ref.at[sl_o])
        pltpu.sync_copy(acc.at[pl.ds(s * WROWS + WCOPY[0] * K, WCOPY[1])],
                        rows_0.at[pl.ds(0, WCOPY[1])])
        pltpu.sync_copy(rows_0.at[pl.ds(0, WCOPY[1])],
                        out_ref.at[pl.ds(c * NODES + s * WROWS + WCOPY[0] * K,
                                         WCOPY[1])])

        @pl.when(s == 0)
        def _():
            pltpu.sync_copy(acc.at[pl.ds(LEFT0, LEFT)],
                            rows_0.at[pl.ds(0, LEFT)])
            pltpu.sync_copy(rows_0.at[pl.ds(0, LEFT)],
                            out_ref.at[pl.ds(c * NODES + LEFT0, LEFT)])

    return deg_kernel, scatter_kernel


R = 1000
GRID = NODES // R


def _ln(xb, g, b):
    mu = jnp.mean(xb, axis=-1, keepdims=True)
    xc = xb - mu
    var = jnp.mean(xc * xc, axis=-1, keepdims=True)
    return xc * lax.rsqrt(var + 1e-5) * g + b


def _inv_sqrt_deg(deg_blk):
    return lax.rsqrt(jnp.maximum(deg_blk, 1.0))


_VEC_SPEC = pl.BlockSpec((1, FEAT), lambda i: (0, 0))
_DEG_SPEC = pl.BlockSpec((R, 1), lambda i: (i, 0))
_MH_SPEC = pl.BlockSpec((2, R, HALF), lambda i: (0, i, 0))
_MH_SHAPE = jax.ShapeDtypeStruct((2, NODES, HALF), jnp.float32)


def _store_halves(m, m_ref):
    m_ref[0] = m[:, :HALF]
    m_ref[1] = m[:, HALF:]


def _tc_pre_body(x_ref, od_ref, g1_ref, b1_ref, m_ref):
    hn = _ln(x_ref[...], g1_ref[...], b1_ref[...])
    _store_halves(hn * _inv_sqrt_deg(od_ref[...]), m_ref)


_tc_pre = pl.pallas_call(
    _tc_pre_body,
    grid=(GRID,),
    in_specs=[
        pl.BlockSpec((R, FEAT), lambda i: (i, 0)),
        _DEG_SPEC,
        _VEC_SPEC,
        _VEC_SPEC,
    ],
    out_specs=_MH_SPEC,
    out_shape=_MH_SHAPE,
)


def _bdot(x, w_ref):
    return jnp.dot(x.astype(jnp.bfloat16), w_ref[...],
                   preferred_element_type=jnp.float32)


def _dense_layer(agg_ref, ind_ref, wg_ref, bg_ref, g2_ref, b2_ref,
                 w1_ref, b1f_ref, w2_ref, b2f_ref):
    a = jnp.concatenate([agg_ref[0], agg_ref[1]], axis=-1)
    a = a * _inv_sqrt_deg(ind_ref[...])
    h = _bdot(a, wg_ref) + bg_ref[...]
    hn2 = _ln(h, g2_ref[...], b2_ref[...])
    t = jnp.maximum(_bdot(hn2, w1_ref) + b1f_ref[...], 0.0)
    return _bdot(t, w2_ref) + b2f_ref[...] + h


def _tc_mid_body(agg_ref, ind_ref, od_ref, wg_ref, bg_ref, g2_ref, b2_ref,
                 w1_ref, b1f_ref, w2_ref, b2f_ref, g1n_ref, b1n_ref, m_ref):
    h = _dense_layer(agg_ref, ind_ref, wg_ref, bg_ref, g2_ref, b2_ref,
                     w1_ref, b1f_ref, w2_ref, b2f_ref)
    hn = _ln(h, g1n_ref[...], b1n_ref[...])
    _store_halves(hn * _inv_sqrt_deg(od_ref[...]), m_ref)


_tc_mid = pl.pallas_call(
    _tc_mid_body,
    grid=(GRID,),
    in_specs=[
        _MH_SPEC,
        _DEG_SPEC,
        _DEG_SPEC,
        pl.BlockSpec((FEAT, FEAT), lambda i: (0, 0)),
        _VEC_SPEC,
        _VEC_SPEC,
        _VEC_SPEC,
        pl.BlockSpec((FEAT, 2 * FEAT), lambda i: (0, 0)),
        pl.BlockSpec((1, 2 * FEAT), lambda i: (0, 0)),
        pl.BlockSpec((2 * FEAT, FEAT), lambda i: (0, 0)),
        _VEC_SPEC,
        _VEC_SPEC,
        _VEC_SPEC,
    ],
    out_specs=_MH_SPEC,
    out_shape=_MH_SHAPE,
)


def _tc_fin_body(agg_ref, ind_ref, wg_ref, bg_ref, g2_ref, b2_ref,
                 w1_ref, b1f_ref, w2_ref, b2f_ref, fni_ref, out_ref):
    h = _dense_layer(agg_ref, ind_ref, wg_ref, bg_ref, g2_ref, b2_ref,
                     w1_ref, b1f_ref, w2_ref, b2f_ref)
    i = pl.program_id(0)
    rows = lax.broadcasted_iota(jnp.int32, (BATCH, R), 1) + i * R
    onehot = (rows == fni_ref[...]).astype(jnp.float32)
    contrib = jnp.dot(onehot, h, preferred_element_type=jnp.float32)

    @pl.when(i == 0)
    def _():
        out_ref[...] = contrib

    @pl.when(i > 0)
    def _():
        out_ref[...] += contrib


_W_SPECS = [
    pl.BlockSpec((FEAT, FEAT), lambda i: (0, 0)),
    _VEC_SPEC,
    _VEC_SPEC,
    _VEC_SPEC,
    pl.BlockSpec((FEAT, 2 * FEAT), lambda i: (0, 0)),
    pl.BlockSpec((1, 2 * FEAT), lambda i: (0, 0)),
    pl.BlockSpec((2 * FEAT, FEAT), lambda i: (0, 0)),
    _VEC_SPEC,
]

_tc_fin = pl.pallas_call(
    _tc_fin_body,
    grid=(GRID,),
    in_specs=[_MH_SPEC, _DEG_SPEC] + _W_SPECS
    + [pl.BlockSpec((BATCH, 1), lambda i: (0, 0))],
    out_specs=pl.BlockSpec((BATCH, FEAT), lambda i: (0, 0)),
    out_shape=jax.ShapeDtypeStruct((BATCH, FEAT), jnp.float32),
)


def kernel(x, edge_index, first_nodes_idx, ln1_g, ln1_b, Wg, bg, ln2_g,
           ln2_b, W1, b1, W2, b2):
    deg_kernel, scatter_kernel = _sc_kernels()
    edge_flat = edge_index.reshape(2 * EDGES)
    deg = deg_kernel(edge_flat)
    od = deg[:NODES].reshape(NODES, 1)
    ind = deg[NPAD:NPAD + NODES].reshape(NODES, 1)

    m = _tc_pre(x, od, ln1_g[0:1], ln1_b[0:1])
    agg = scatter_kernel(m.reshape(2 * NODES, HALF), edge_flat)
    Wgh = Wg.astype(jnp.bfloat16)
    W1h = W1.astype(jnp.bfloat16)
    W2h = W2.astype(jnp.bfloat16)
    m = _tc_mid(agg.reshape(2, NODES, HALF), ind, od, Wgh[0],
                bg[0:1], ln2_g[0:1], ln2_b[0:1], W1h[0], b1[0:1],
                W2h[0], b2[0:1], ln1_g[1:2], ln1_b[1:2])
    agg = scatter_kernel(m.reshape(2 * NODES, HALF), edge_flat)
    return _tc_fin(agg.reshape(2, NODES, HALF), ind, Wgh[1],
                   bg[1:2], ln2_g[1:2], ln2_b[1:2], W1h[1], b1[1:2],
                   W2h[1], b2[1:2], first_nodes_idx.reshape(BATCH, 1))
